# baseline (speedup 1.0000x reference)
import functools

import jax
import jax.numpy as jnp
from jax import lax
from jax.experimental import pallas as pl
from jax.experimental.pallas import tpu as pltpu
from jax.experimental.pallas import tpu_sc as plsc

EPS = 1e-5
NC = 2
NS = 16
L = 16
CH = 64
FRAC0 = 0.55


def _zero_fill(ref, nrows, ncolgroups):
    zeros = jnp.zeros((L,), jnp.float32)

    def body(r, _):
        for cg in range(ncolgroups):
            ref[r, pl.ds(cg * L, L)] = zeros
        return 0

    lax.fori_loop(0, nrows, body, 0)


def _rows_per_tile(n_nodes):
    rpt = -(-n_nodes // NS)
    return ((rpt + 7) // 8) * 8


def _zero_acc(zbuf, acc_sh, base, rpt):
    nfull, rem = divmod(rpt, CH)
    for k in range(nfull):
        pltpu.sync_copy(zbuf.at[pl.ds(0, CH)],
                        acc_sh.at[pl.ds(base + k * CH, CH)])
    if rem:
        pltpu.sync_copy(zbuf.at[pl.ds(0, rem)],
                        acc_sh.at[pl.ds(base + nfull * CH, rem)])


def _make_deg_kernel(n_nodes, nch, d):
    mesh = plsc.VectorSubcoreMesh(core_axis_name="c", subcore_axis_name="s")
    rpt = _rows_per_tile(n_nodes)
    n_pad = rpt * NS

    @functools.partial(
        pl.kernel,
        mesh=mesh,
        out_type=jax.ShapeDtypeStruct((NC, n_pad, d), jnp.float32),
        scratch_types=(
            [pltpu.VMEM((CH,), jnp.int32) for _ in range(4)]
            + [pltpu.VMEM((CH, L), jnp.float32) for _ in range(4)]
            + [pltpu.VMEM((CH, d), jnp.float32)]
            + [pltpu.SemaphoreType.DMA for _ in range(4)]
            + [pltpu.VMEM_SHARED((n_pad, d), jnp.float32)]
        ),
    )
    def deg_kernel(dst_hbm, ewx_hbm, out_hbm, *refs):
        db = refs[0:4]
        eb = refs[4:8]
        rows_v = refs[8]
        lsem = refs[9:13]
        acc_sh = refs[13]

        c = lax.axis_index("c")
        s = lax.axis_index("s")
        w = s * NC + c
        base = s * rpt
        start = w * nch

        def small_load(k, j):
            pltpu.async_copy(dst_hbm.at[start + j], db[k], lsem[k])
            pltpu.async_copy(ewx_hbm.at[start + j], eb[k], lsem[k])

        def small_wait(k, j):
            pltpu.make_async_copy(dst_hbm.at[start + j], db[k],
                                  lsem[k]).wait()
            pltpu.make_async_copy(ewx_hbm.at[start + j], eb[k],
                                  lsem[k]).wait()

        _zero_fill(rows_v, CH, d // L)
        _zero_acc(rows_v, acc_sh, base, rpt)
        plsc.subcore_barrier()

        pltpu.sync_copy(dst_hbm.at[start], db[0])
        pltpu.sync_copy(ewx_hbm.at[start], eb[0])
        small_load(1, 1)

        def quad(j4, _):
            for u in range(4):
                k = u
                kl = (u + 2) % 4
                j = j4 * 4 + u

                @pl.when(j >= 1)
                def _():
                    small_wait(k, j)

                @pl.when(j + 2 < nch)
                def _():
                    small_load(kl, j + 2)

                def row(r, _):
                    rows_v[r, pl.ds(0, L)] = eb[k][r]
                    return 0

                lax.fori_loop(0, CH, row, 0)
                pltpu.sync_copy(rows_v, acc_sh.at[db[k]], add=True)
            return 0

        lax.fori_loop(0, nch // 4, quad, 0)
        plsc.subcore_barrier()
        pltpu.sync_copy(acc_sh.at[pl.ds(base, rpt)],
                        out_hbm.at[c, pl.ds(base, rpt)])

    return deg_kernel


def _make_scatter_kernel(n_nodes, q0, q1, d):
    assert q0 % 4 == 0 and q1 % 4 == 0
    mesh = plsc.VectorSubcoreMesh(core_axis_name="c", subcore_axis_name="s")
    rpt = _rows_per_tile(n_nodes)
    n_pad = rpt * NS
    ncg = d // L

    @functools.partial(
        pl.kernel,
        mesh=mesh,
        out_type=jax.ShapeDtypeStruct((NC, n_pad, d), jnp.float32),
        scratch_types=(
            [pltpu.VMEM((CH,), jnp.int32) for _ in range(4)]
            + [pltpu.VMEM((CH,), jnp.int32) for _ in range(4)]
            + [pltpu.VMEM((CH, L), jnp.float32) for _ in range(4)]
            + [pltpu.VMEM((CH, d), jnp.float32) for _ in range(2)]
            + [pltpu.SemaphoreType.DMA for _ in range(8)]
            + [pltpu.VMEM_SHARED((n_pad, d), jnp.float32)]
        ),
    )
    def scat_kernel(g_hbm, src_hbm, dst_hbm, ewx_hbm, out_hbm, *refs):
        sb = refs[0:4]
        db = refs[4:8]
        eb = refs[8:12]
        rows = refs[12:14]
        lsem = refs[14:18]
        gsem = refs[18:20]
        ssem = refs[20:22]
        acc_sh = refs[22]

        c = lax.axis_index("c")
        s = lax.axis_index("s")
        base = s * rpt
        qc = jnp.where(c == 0, q0, q1)
        start = jnp.where(c == 0, s * q0, NS * q0 + s * q1)

        def small_load(k, j):
            pltpu.async_copy(src_hbm.at[start + j], sb[k], lsem[k])
            pltpu.async_copy(dst_hbm.at[start + j], db[k], lsem[k])
            pltpu.async_copy(ewx_hbm.at[start + j], eb[k], lsem[k])

        def small_wait(k, j):
            pltpu.make_async_copy(src_hbm.at[start + j], sb[k],
                                  lsem[k]).wait()
            pltpu.make_async_copy(dst_hbm.at[start + j], db[k],
                                  lsem[k]).wait()
            pltpu.make_async_copy(ewx_hbm.at[start + j], eb[k],
                                  lsem[k]).wait()

        def gather_start(b, k):
            pltpu.async_copy(g_hbm.at[sb[k]], rows[b], gsem[b])

        def gather_wait(b, k):
            pltpu.make_async_copy(g_hbm.at[sb[k]], rows[b], gsem[b]).wait()

        def scat_start(b, k):
            pltpu.async_copy(rows[b], acc_sh.at[db[k]], ssem[b], add=True)

        def scat_wait(b, k):
            pltpu.make_async_copy(rows[b], acc_sh.at[db[k]], ssem[b]).wait()

        def scale(b, k):
            def row(r2, _):
                for u in range(2):
                    r = r2 * 2 + u
                    ewb = eb[k][r]
                    for cg in range(ncg):
                        rows[b][r, pl.ds(cg * L, L)] = (
                            rows[b][r, pl.ds(cg * L, L)] * ewb)
                return 0

            lax.fori_loop(0, CH // 2, row, 0)

        _zero_fill(rows[0], CH, ncg)
        _zero_acc(rows[0], acc_sh, base, rpt)
        plsc.subcore_barrier()

        pltpu.sync_copy(src_hbm.at[start], sb[0])
        pltpu.sync_copy(dst_hbm.at[start], db[0])
        pltpu.sync_copy(ewx_hbm.at[start], eb[0])
        gather_start(0, 0)
        small_load(1, 1)

        def quad(j4, _):
            for u in range(4):
                b = u % 2
                o = 1 - b
                k = u
                ko = (u + 1) % 4
                kl = (u + 2) % 4
                j = j4 * 4 + u
                gather_wait(b, k)

                @pl.when(j + 1 < qc)
                def _():
                    small_wait(ko, j + 1)
                    gather_start(o, ko)

                @pl.when(j + 2 < qc)
                def _():
                    small_load(kl, j + 2)

                scale(b, k)
                pltpu.sync_copy(rows[b], acc_sh.at[db[k]], add=True)
            return 0

        lax.fori_loop(0, qc // 4, quad, 0)
        plsc.subcore_barrier()
        pltpu.sync_copy(acc_sh.at[pl.ds(base, rpt)],
                        out_hbm.at[c, pl.ds(base, rpt)])

    return scat_kernel


def _tc_call(body, out_shapes):
    return pl.pallas_call(
        body,
        out_shape=[jax.ShapeDtypeStruct(s, jnp.float32) for s in out_shapes],
    )


def _tck1_body(n, x_ref, w_ref, degp_ref, dis_ref, g_ref):
    deg = degp_ref[0, 0:n, 0:1] + degp_ref[1, 0:n, 0:1] + 1.0
    dis = jnp.where(deg > 0, lax.rsqrt(deg), 0.0)
    dis_ref[...] = dis
    m = jnp.dot(x_ref[...], w_ref[...], preferred_element_type=jnp.float32)
    g_ref[...] = m * dis


def _tck_mid_body(n, s_ref, g_ref, dis_ref, b_ref, gam_ref, bet_ref, w_ref,
                  gnext_ref):
    dis = dis_ref[...]
    t = (s_ref[0, 0:n, :] + s_ref[1, 0:n, :] + g_ref[...]) * dis \
        + b_ref[...][None, :]
    mu = jnp.mean(t, axis=0, keepdims=True)
    var = jnp.mean((t - mu) ** 2, axis=0, keepdims=True)
    h = (t - mu) * lax.rsqrt(var + EPS) * gam_ref[...][None, :] \
        + bet_ref[...][None, :]
    h = jnp.maximum(h, 0.0)
    m = jnp.dot(h, w_ref[...], preferred_element_type=jnp.float32)
    gnext_ref[...] = m * dis


def _tck_final_body(n, d_out, s_ref, g_ref, dis_ref, b_ref, out_ref):
    o = (s_ref[0, 0:n, 0:d_out] + s_ref[1, 0:n, 0:d_out]
         + g_ref[..., 0:d_out]) * dis_ref[...] + b_ref[...][None, :]
    o = o - jnp.max(o, axis=-1, keepdims=True)
    out_ref[...] = o - jnp.log(jnp.sum(jnp.exp(o), axis=-1, keepdims=True))


def kernel(x, edge_index, edge_weight, W0, b0, g0, be0, W1, b1, g1, be1,
           W2, b2):
    n = x.shape[0]
    d_in = x.shape[1]
    d_h = W0.shape[1]
    d_out = W2.shape[1]
    e = edge_index.shape[1]

    nw = NC * NS
    qp = -(-e // (CH * NS))
    qp = ((qp + 7) // 8) * 8
    q0 = int(round(qp * FRAC0 / 4.0)) * 4
    q0 = min(max(q0, 4), qp - 4)
    q1 = qp - q0
    g_chunks = NS * qp
    e_pad = g_chunks * CH
    pad = e_pad - e
    nch = g_chunks // nw

    spread = (jnp.arange(pad, dtype=jnp.int32) * 8) % n
    src = jnp.concatenate([edge_index[0], spread])
    dst = jnp.concatenate([edge_index[1], spread])
    ewp = jnp.pad(edge_weight, (0, pad))
    src3 = src.reshape(g_chunks, CH)
    dst3 = dst.reshape(g_chunks, CH)
    ewx = jnp.broadcast_to(ewp[:, None], (e_pad, L)).reshape(g_chunks, CH, L)

    degp = _make_deg_kernel(n, nch, d_h)(dst3, ewx)
    dis, gg0 = _tc_call(functools.partial(_tck1_body, n),
                        [(n, 1), (n, d_h)])(x, W0, degp)

    scat_h = _make_scatter_kernel(n, q0, q1, d_h)
    mid = functools.partial(_tck_mid_body, n)
    fin = functools.partial(_tck_final_body, n, d_out)
    W2p = jnp.pad(W2, ((0, 0), (0, d_h - d_out)))

    s0 = scat_h(gg0, src3, dst3, ewx)
    (gg1,) = _tc_call(mid, [(n, d_h)])(s0, gg0, dis, b0, g0, be0, W1)
    s1 = scat_h(gg1, src3, dst3, ewx)
    (gg2,) = _tc_call(mid, [(n, d_h)])(s1, gg1, dis, b1, g1, be1, W2p)
    s2 = scat_h(gg2, src3, dst3, ewx)
    (out,) = _tc_call(fin, [(n, d_out)])(s2, gg2, dis, b2)
    return out


# speedup vs baseline: 1.0634x; 1.0634x over previous
import functools

import jax
import jax.numpy as jnp
from jax import lax
from jax.experimental import pallas as pl
from jax.experimental.pallas import tpu as pltpu
from jax.experimental.pallas import tpu_sc as plsc

EPS = 1e-5
NC = 2
NS = 16
L = 16
CH = 64
FRAC0 = 0.5


def _zero_fill(ref, nrows, ncolgroups):
    zeros = jnp.zeros((L,), jnp.float32)

    def body(r, _):
        for cg in range(ncolgroups):
            ref[r, pl.ds(cg * L, L)] = zeros
        return 0

    lax.fori_loop(0, nrows, body, 0)


def _rows_per_tile(n_nodes):
    rpt = -(-n_nodes // NS)
    return ((rpt + 7) // 8) * 8


def _zero_acc(zbuf, acc_sh, base, rpt):
    nfull, rem = divmod(rpt, CH)
    for k in range(nfull):
        pltpu.sync_copy(zbuf.at[pl.ds(0, CH)],
                        acc_sh.at[pl.ds(base + k * CH, CH)])
    if rem:
        pltpu.sync_copy(zbuf.at[pl.ds(0, rem)],
                        acc_sh.at[pl.ds(base + nfull * CH, rem)])


def _make_deg_kernel(n_nodes, nch, d):
    mesh = plsc.VectorSubcoreMesh(core_axis_name="c", subcore_axis_name="s")
    rpt = _rows_per_tile(n_nodes)
    n_pad = rpt * NS

    @functools.partial(
        pl.kernel,
        mesh=mesh,
        out_type=jax.ShapeDtypeStruct((NC, n_pad, d), jnp.float32),
        scratch_types=(
            [pltpu.VMEM((CH,), jnp.int32) for _ in range(4)]
            + [pltpu.VMEM((CH, L), jnp.float32) for _ in range(4)]
            + [pltpu.VMEM((CH, d), jnp.float32)]
            + [pltpu.SemaphoreType.DMA for _ in range(4)]
            + [pltpu.VMEM_SHARED((n_pad, d), jnp.float32)]
        ),
    )
    def deg_kernel(dst_hbm, ewx_hbm, out_hbm, *refs):
        db = refs[0:4]
        eb = refs[4:8]
        rows_v = refs[8]
        lsem = refs[9:13]
        acc_sh = refs[13]

        c = lax.axis_index("c")
        s = lax.axis_index("s")
        w = s * NC + c
        base = s * rpt
        start = w * nch

        def small_load(k, j):
            pltpu.async_copy(dst_hbm.at[start + j], db[k], lsem[k])
            pltpu.async_copy(ewx_hbm.at[start + j], eb[k], lsem[k])

        def small_wait(k, j):
            pltpu.make_async_copy(dst_hbm.at[start + j], db[k],
                                  lsem[k]).wait()
            pltpu.make_async_copy(ewx_hbm.at[start + j], eb[k],
                                  lsem[k]).wait()

        _zero_fill(rows_v, CH, d // L)
        _zero_acc(rows_v, acc_sh, base, rpt)
        plsc.subcore_barrier()

        pltpu.sync_copy(dst_hbm.at[start], db[0])
        pltpu.sync_copy(ewx_hbm.at[start], eb[0])
        small_load(1, 1)

        def quad(j4, _):
            for u in range(4):
                k = u
                kl = (u + 2) % 4
                j = j4 * 4 + u

                @pl.when(j >= 1)
                def _():
                    small_wait(k, j)

                @pl.when(j + 2 < nch)
                def _():
                    small_load(kl, j + 2)

                def row(r, _):
                    rows_v[r, pl.ds(0, L)] = eb[k][r]
                    return 0

                lax.fori_loop(0, CH, row, 0)
                pltpu.sync_copy(rows_v, acc_sh.at[db[k]], add=True)
            return 0

        lax.fori_loop(0, nch // 4, quad, 0)
        plsc.subcore_barrier()
        pltpu.sync_copy(acc_sh.at[pl.ds(base, rpt)],
                        out_hbm.at[c, pl.ds(base, rpt)])

    return deg_kernel


def _make_scatter_kernel(n_nodes, q0, q1, d):
    assert q0 % 4 == 0 and q1 % 4 == 0
    mesh = plsc.VectorSubcoreMesh(core_axis_name="c", subcore_axis_name="s")
    rpt = _rows_per_tile(n_nodes)
    n_pad = rpt * NS
    ncg = d // L

    @functools.partial(
        pl.kernel,
        mesh=mesh,
        out_type=jax.ShapeDtypeStruct((NC, n_pad, d), jnp.float32),
        scratch_types=(
            [pltpu.VMEM((CH,), jnp.int32) for _ in range(4)]
            + [pltpu.VMEM((CH,), jnp.int32) for _ in range(4)]
            + [pltpu.VMEM((CH, L), jnp.float32) for _ in range(4)]
            + [pltpu.VMEM((CH, d), jnp.float32) for _ in range(2)]
            + [pltpu.SemaphoreType.DMA for _ in range(8)]
            + [pltpu.VMEM_SHARED((n_pad, d), jnp.float32)]
        ),
    )
    def scat_kernel(g_hbm, src_hbm, dst_hbm, ewx_hbm, out_hbm, *refs):
        sb = refs[0:4]
        db = refs[4:8]
        eb = refs[8:12]
        rows = refs[12:14]
        lsem = refs[14:18]
        gsem = refs[18:20]
        ssem = refs[20:22]
        acc_sh = refs[22]

        c = lax.axis_index("c")
        s = lax.axis_index("s")
        base = s * rpt
        qc = jnp.where(c == 0, q0, q1)
        start = jnp.where(c == 0, s * q0, NS * q0 + s * q1)

        def small_load(k, j):
            pltpu.async_copy(src_hbm.at[start + j], sb[k], lsem[k])
            pltpu.async_copy(dst_hbm.at[start + j], db[k], lsem[k])
            pltpu.async_copy(ewx_hbm.at[start + j], eb[k], lsem[k])

        def small_wait(k, j):
            pltpu.make_async_copy(src_hbm.at[start + j], sb[k],
                                  lsem[k]).wait()
            pltpu.make_async_copy(dst_hbm.at[start + j], db[k],
                                  lsem[k]).wait()
            pltpu.make_async_copy(ewx_hbm.at[start + j], eb[k],
                                  lsem[k]).wait()

        def gather_start(b, k):
            pltpu.async_copy(g_hbm.at[sb[k]], rows[b], gsem[b])

        def gather_wait(b, k):
            pltpu.make_async_copy(g_hbm.at[sb[k]], rows[b], gsem[b]).wait()

        def scat_start(b, k):
            pltpu.async_copy(rows[b], acc_sh.at[db[k]], ssem[b], add=True)

        def scat_wait(b, k):
            pltpu.make_async_copy(rows[b], acc_sh.at[db[k]], ssem[b]).wait()

        def scale(b, k):
            def row(r2, _):
                for u in range(2):
                    r = r2 * 2 + u
                    ewb = eb[k][r]
                    for cg in range(ncg):
                        rows[b][r, pl.ds(cg * L, L)] = (
                            rows[b][r, pl.ds(cg * L, L)] * ewb)
                return 0

            lax.fori_loop(0, CH // 2, row, 0)

        _zero_fill(rows[0], CH, ncg)
        _zero_acc(rows[0], acc_sh, base, rpt)
        plsc.subcore_barrier()

        pltpu.sync_copy(src_hbm.at[start], sb[0])
        pltpu.sync_copy(dst_hbm.at[start], db[0])
        pltpu.sync_copy(ewx_hbm.at[start], eb[0])
        gather_start(0, 0)
        small_load(1, 1)

        def quad(j4, _):
            for u in range(4):
                b = u % 2
                o = 1 - b
                k = u
                ko = (u + 1) % 4
                kl = (u + 2) % 4
                j = j4 * 4 + u
                gather_wait(b, k)

                @pl.when(j + 1 < qc)
                def _():
                    small_wait(ko, j + 1)
                    gather_start(o, ko)

                @pl.when(j + 2 < qc)
                def _():
                    small_load(kl, j + 2)

                scale(b, k)
                pltpu.sync_copy(rows[b], acc_sh.at[db[k]], add=True)
            return 0

        lax.fori_loop(0, qc // 4, quad, 0)
        plsc.subcore_barrier()
        pltpu.sync_copy(acc_sh.at[pl.ds(base, rpt)],
                        out_hbm.at[c, pl.ds(base, rpt)])

    return scat_kernel


def _tc_call(body, out_shapes):
    return pl.pallas_call(
        body,
        out_shape=[jax.ShapeDtypeStruct(s, jnp.float32) for s in out_shapes],
    )


def _tck1_body(n, x_ref, w_ref, degp_ref, dis_ref, g_ref):
    deg = degp_ref[0, 0:n, 0:1] + degp_ref[1, 0:n, 0:1] + 1.0
    dis = jnp.where(deg > 0, lax.rsqrt(deg), 0.0)
    dis_ref[...] = dis
    m = jnp.dot(x_ref[...], w_ref[...], preferred_element_type=jnp.float32)
    g_ref[...] = m * dis


def _tck_mid_body(n, s_ref, g_ref, dis_ref, b_ref, gam_ref, bet_ref, w_ref,
                  gnext_ref):
    dis = dis_ref[...]
    t = (s_ref[0, 0:n, :] + s_ref[1, 0:n, :] + g_ref[...]) * dis \
        + b_ref[...][None, :]
    mu = jnp.mean(t, axis=0, keepdims=True)
    var = jnp.mean((t - mu) ** 2, axis=0, keepdims=True)
    h = (t - mu) * lax.rsqrt(var + EPS) * gam_ref[...][None, :] \
        + bet_ref[...][None, :]
    h = jnp.maximum(h, 0.0)
    m = jnp.dot(h, w_ref[...], preferred_element_type=jnp.float32)
    gnext_ref[...] = m * dis


def _tck_final_body(n, d_out, s_ref, g_ref, dis_ref, b_ref, out_ref):
    o = (s_ref[0, 0:n, 0:d_out] + s_ref[1, 0:n, 0:d_out]
         + g_ref[..., 0:d_out]) * dis_ref[...] + b_ref[...][None, :]
    o = o - jnp.max(o, axis=-1, keepdims=True)
    out_ref[...] = o - jnp.log(jnp.sum(jnp.exp(o), axis=-1, keepdims=True))


def kernel(x, edge_index, edge_weight, W0, b0, g0, be0, W1, b1, g1, be1,
           W2, b2):
    n = x.shape[0]
    d_in = x.shape[1]
    d_h = W0.shape[1]
    d_out = W2.shape[1]
    e = edge_index.shape[1]

    nw = NC * NS
    qp = -(-e // (CH * NS))
    qp = ((qp + 7) // 8) * 8
    q0 = int(round(qp * FRAC0 / 4.0)) * 4
    q0 = min(max(q0, 4), qp - 4)
    q1 = qp - q0
    g_chunks = NS * qp
    e_pad = g_chunks * CH
    pad = e_pad - e
    nch = g_chunks // nw

    spread = (jnp.arange(pad, dtype=jnp.int32) * 8) % n
    src = jnp.concatenate([edge_index[0], spread])
    dst = jnp.concatenate([edge_index[1], spread])
    ewp = jnp.pad(edge_weight, (0, pad))
    src3 = src.reshape(g_chunks, CH)
    dst3 = dst.reshape(g_chunks, CH)
    ewx = jnp.broadcast_to(ewp[:, None], (e_pad, L)).reshape(g_chunks, CH, L)

    degp = _make_deg_kernel(n, nch, d_h)(dst3, ewx)
    dis, gg0 = _tc_call(functools.partial(_tck1_body, n),
                        [(n, 1), (n, d_h)])(x, W0, degp)

    scat_h = _make_scatter_kernel(n, q0, q1, d_h)
    mid = functools.partial(_tck_mid_body, n)
    fin = functools.partial(_tck_final_body, n, d_out)
    W2p = jnp.pad(W2, ((0, 0), (0, d_h - d_out)))

    s0 = scat_h(gg0, src3, dst3, ewx)
    (gg1,) = _tc_call(mid, [(n, d_h)])(s0, gg0, dis, b0, g0, be0, W1)
    s1 = scat_h(gg1, src3, dst3, ewx)
    (gg2,) = _tc_call(mid, [(n, d_h)])(s1, gg1, dis, b1, g1, be1, W2p)
    s2 = scat_h(gg2, src3, dst3, ewx)
    (out,) = _tc_call(fin, [(n, d_out)])(s2, gg2, dis, b2)
    return out
